# baseline (speedup 1.0000x reference)
import functools

import jax
import jax.numpy as jnp
from jax import lax
from jax.experimental import pallas as pl
from jax.experimental.pallas import tpu as pltpu
from jax.experimental.pallas import tpu_sc as plsc

N = 10000
E = 320000
C = 128
NPAD = 10240
NC = 2
NS = 16
NW = NC * NS
CH = 80
NROWS = E // CH
L = 16
NBUF = 5

_mesh = lambda: plsc.VectorSubcoreMesh(core_axis_name="c", subcore_axis_name="s")
_params = lambda: pltpu.CompilerParams(needs_layout_passes=False)


def _rsqrt_newton(d):
    i = plsc.bitcast(d, jnp.int32)
    y = plsc.bitcast(jnp.int32(0x5F3759DF) - lax.shift_right_logical(i, 1),
                     jnp.float32)
    for _ in range(3):
        y = y * (1.5 - 0.5 * d * y * y)
    return jnp.where(d > 0, y, 0.0)


BCH = 25
EPB = BCH * CH
NBLK = E // EPB


def _agg_pipeline(tab_h, rowb_v, col2_v, normb_v, bufs, gsems, ssems, acc_s):
    def fire_gather(q, b):
        pltpu.async_copy(tab_h.at[rowb_v.at[pl.ds(q * CH, CH)]], bufs[b],
                         gsems[b])

    def wait_gather(q, b):
        pltpu.make_async_copy(tab_h.at[rowb_v.at[pl.ds(q * CH, CH)]],
                              bufs[b], gsems[b]).wait()

    def wait_scatter(q, b):
        pltpu.make_async_copy(bufs[b], acc_s.at[col2_v.at[q]],
                              ssems[b]).wait()

    fire_gather(0, 0)
    fire_gather(1, 1)

    def slot(q, b, g):
        wait_gather(q, b)

        for g2 in range(CH // L):
            nv = normb_v[pl.ds(q * CH + g2 * L, L)]
            for e in range(L):
                sv = jnp.full((L,), nv[e], jnp.float32)
                r = g2 * L + e
                for u in range(C // L):
                    bufs[b][r, pl.ds(u * L, L)] = (
                        bufs[b][r, pl.ds(u * L, L)] * sv)

        pltpu.async_copy(bufs[b], acc_s.at[col2_v.at[q]], ssems[b], add=True)

        bg = (b + 2) % 3

        def refill():
            wait_scatter(q, bg)
            fire_gather(q + 2, bg)

        if b == 0:
            @pl.when(g > 0)
            def _():
                refill()

            @pl.when(g == 0)
            def _():
                fire_gather(q + 2, bg)
        elif b == 1:
            refill()
        else:
            @pl.when(g < BCH // 3 - 1)
            def _():
                refill()

    def outer_body(g, _):
        for b in range(3):
            slot(g * 3 + b, b, g)
        return 0
    lax.fori_loop(0, BCH // 3, outer_body, 0)

    slot_q = BCH - 1
    wait_gather(slot_q, 0)

    for g2 in range(CH // L):
        nv = normb_v[pl.ds(slot_q * CH + g2 * L, L)]
        for e in range(L):
            sv = jnp.full((L,), nv[e], jnp.float32)
            r = g2 * L + e
            for u in range(C // L):
                bufs[0][r, pl.ds(u * L, L)] = bufs[0][r, pl.ds(u * L, L)] * sv
    pltpu.async_copy(bufs[0], acc_s.at[col2_v.at[slot_q]], ssems[0], add=True)

    for b in range(3):
        wait_scatter(0, b)


def _agg_xh_kernel(rowf_h, row3_h, col3_h, ewf_h, xh_h, zeros_h,
                   a_out_h, norm_out_h,
                   dis_v, rowb_v, col2_v, normb_v,
                   b0, b1, b2,
                   acc_s, deg_s,
                   g0, g1, g2, s0, s1, s2, dsem):
    c = lax.axis_index("c")
    s = lax.axis_index("s")
    bufs = (b0, b1, b2)
    gsems = (g0, g1, g2)
    ssems = (s0, s1, s2)

    def zb(i, _):
        normb_v[pl.ds(i * L, L)] = jnp.zeros((L,), jnp.float32)
        return 0
    lax.fori_loop(0, 640 // L, zb, 0)

    @pl.when(s < 15)
    def _():
        pltpu.sync_copy(normb_v.at[pl.ds(0, 640)],
                        deg_s.at[pl.ds(s * 640, 640)])
        for i in range(5):
            pltpu.sync_copy(zeros_h, acc_s.at[pl.ds(s * 640 + i * 128, 128)])

    @pl.when(s == 15)
    def _():
        pltpu.sync_copy(normb_v.at[pl.ds(0, 400)],
                        deg_s.at[pl.ds(9600, 400)])
        for i in range(3):
            pltpu.sync_copy(zeros_h, acc_s.at[pl.ds(9600 + i * 128, 128)])
        pltpu.sync_copy(zeros_h.at[pl.ds(0, 16)],
                        acc_s.at[pl.ds(9984, 16)])
    plsc.subcore_barrier()

    def deg_blk(kb, _):
        k2 = s * (2 * NBLK // NW) + kb
        pltpu.async_copy(row3_h.at[k2], col2_v, g0)
        pltpu.async_copy(ewf_h.at[k2], normb_v, g1)
        pltpu.make_async_copy(row3_h.at[k2], col2_v, g0).wait()
        pltpu.make_async_copy(ewf_h.at[k2], normb_v, g1).wait()

        def dfire(q, _):
            pltpu.async_copy(normb_v.at[pl.ds(q * CH, CH)],
                             deg_s.at[col2_v.at[q]], dsem, add=True)
            return 0
        lax.fori_loop(0, BCH, dfire, 0)

        def ddrain(q, _):
            pltpu.make_async_copy(normb_v.at[pl.ds(q * CH, CH)],
                                  deg_s.at[col2_v.at[q]], dsem).wait()
            return 0
        lax.fori_loop(0, BCH, ddrain, 0)
        return 0
    lax.fori_loop(0, 2 * NBLK // NW, deg_blk, 0)
    plsc.subcore_barrier()

    def newton(i, _):
        d = normb_v[pl.ds(i * L, L)]
        normb_v[pl.ds(i * L, L)] = _rsqrt_newton(d)
        return 0

    @pl.when(s < 15)
    def _():
        pltpu.sync_copy(deg_s.at[pl.ds(s * 640, 640)],
                        normb_v.at[pl.ds(0, 640)])
        lax.fori_loop(0, 640 // L, newton, 0)
        pltpu.sync_copy(normb_v.at[pl.ds(0, 640)],
                        deg_s.at[pl.ds(s * 640, 640)])

    @pl.when(s == 15)
    def _():
        pltpu.sync_copy(deg_s.at[pl.ds(9600, 400)],
                        normb_v.at[pl.ds(0, 400)])
        lax.fori_loop(0, 400 // L, newton, 0)
        pltpu.sync_copy(normb_v.at[pl.ds(0, 400)],
                        deg_s.at[pl.ds(9600, 400)])
    plsc.subcore_barrier()
    pltpu.sync_copy(deg_s, dis_v)

    off = jnp.full((L,), c * N, jnp.int32)

    def main_blk(kb, _):
        k2 = s * (2 * NBLK // NW) + kb
        pltpu.async_copy(rowf_h.at[k2], rowb_v, g0)
        pltpu.async_copy(col3_h.at[k2], col2_v, g1)
        pltpu.async_copy(ewf_h.at[k2], normb_v, g2)
        pltpu.make_async_copy(rowf_h.at[k2], rowb_v, g0).wait()
        pltpu.make_async_copy(col3_h.at[k2], col2_v, g1).wait()
        pltpu.make_async_copy(ewf_h.at[k2], normb_v, g2).wait()

        def normk(i, _):
            r16 = rowb_v[pl.ds(i * L, L)]
            c16 = col2_v[i // (CH // L), pl.ds((i % (CH // L)) * L, L)]
            w16 = normb_v[pl.ds(i * L, L)]
            a = plsc.load_gather(dis_v, [r16])
            b = plsc.load_gather(dis_v, [c16])
            normb_v[pl.ds(i * L, L)] = -(a * w16) * b
            rowb_v[pl.ds(i * L, L)] = r16 + off
            return 0
        lax.fori_loop(0, EPB // L, normk, 0)

        @pl.when(c == 0)
        def _():
            pltpu.sync_copy(normb_v, norm_out_h.at[k2])

        _agg_pipeline(xh_h, rowb_v, col2_v, normb_v, bufs, gsems, ssems,
                      acc_s)
        return 0
    lax.fori_loop(0, 2 * NBLK // NW, main_blk, 0)

    plsc.subcore_barrier()

    @pl.when(s < 15)
    def _():
        pltpu.sync_copy(acc_s.at[pl.ds(s * 640, 640)],
                        a_out_h.at[pl.ds(c * N + s * 640, 640)])

    @pl.when(s == 15)
    def _():
        pltpu.sync_copy(acc_s.at[pl.ds(9600, 400)],
                        a_out_h.at[pl.ds(c * N + 9600, 400)])


def _agg_xh(rowf, row3, col3, ewf, xh, zeros):
    f = pl.kernel(
        _agg_xh_kernel,
        out_type=(jax.ShapeDtypeStruct((2 * N, C), jnp.float32),
                  jax.ShapeDtypeStruct((NBLK, EPB), jnp.float32)),
        mesh=_mesh(),
        compiler_params=_params(),
        scratch_types=[
            pltpu.VMEM((N,), jnp.float32),
            pltpu.VMEM((EPB,), jnp.int32),
            pltpu.VMEM((BCH, CH), jnp.int32),
            pltpu.VMEM((EPB,), jnp.float32),
        ] + [pltpu.VMEM((CH, C), jnp.float32)] * 3
        + [pltpu.VMEM_SHARED((N, C), jnp.float32),
           pltpu.VMEM_SHARED((N,), jnp.float32)]
        + [pltpu.SemaphoreType.DMA] * 7,
    )
    return f(rowf, row3, col3, ewf, xh, zeros)


def _agg_rh_kernel(rowf_h, col3_h, normf_h, rh_h, zeros_h, a_out_h,
                   rowb_v, col2_v, normb_v,
                   b0, b1, b2,
                   acc_s,
                   g0, g1, g2, s0, s1, s2):
    c = lax.axis_index("c")
    s = lax.axis_index("s")
    w = c * NS + s
    bufs = (b0, b1, b2)
    gsems = (g0, g1, g2)
    ssems = (s0, s1, s2)

    @pl.when(s < 15)
    def _():
        for i in range(5):
            pltpu.sync_copy(zeros_h, acc_s.at[pl.ds(s * 640 + i * 128, 128)])

    @pl.when(s == 15)
    def _():
        for i in range(3):
            pltpu.sync_copy(zeros_h, acc_s.at[pl.ds(9600 + i * 128, 128)])
        pltpu.sync_copy(zeros_h.at[pl.ds(0, 16)],
                        acc_s.at[pl.ds(9984, 16)])
    plsc.subcore_barrier()

    def main_blk(kb, _):
        k2 = w * (NBLK // NW) + kb
        pltpu.async_copy(rowf_h.at[k2], rowb_v, g0)
        pltpu.async_copy(col3_h.at[k2], col2_v, g1)
        pltpu.async_copy(normf_h.at[k2], normb_v, g2)
        pltpu.make_async_copy(rowf_h.at[k2], rowb_v, g0).wait()
        pltpu.make_async_copy(col3_h.at[k2], col2_v, g1).wait()
        pltpu.make_async_copy(normf_h.at[k2], normb_v, g2).wait()
        _agg_pipeline(rh_h, rowb_v, col2_v, normb_v, bufs, gsems, ssems,
                      acc_s)
        return 0
    lax.fori_loop(0, NBLK // NW, main_blk, 0)

    plsc.subcore_barrier()

    @pl.when(s < 15)
    def _():
        pltpu.sync_copy(acc_s.at[pl.ds(s * 640, 640)],
                        a_out_h.at[pl.ds(c * N + s * 640, 640)])

    @pl.when(s == 15)
    def _():
        pltpu.sync_copy(acc_s.at[pl.ds(9600, 400)],
                        a_out_h.at[pl.ds(c * N + 9600, 400)])


def _agg_rh(rowf, col3, normf, rh, zeros):
    f = pl.kernel(
        _agg_rh_kernel,
        out_type=jax.ShapeDtypeStruct((2 * N, C), jnp.float32),
        mesh=_mesh(),
        compiler_params=_params(),
        scratch_types=[
            pltpu.VMEM((EPB,), jnp.int32),
            pltpu.VMEM((BCH, CH), jnp.int32),
            pltpu.VMEM((EPB,), jnp.float32),
        ] + [pltpu.VMEM((CH, C), jnp.float32)] * 3
        + [pltpu.VMEM_SHARED((N, C), jnp.float32)]
        + [pltpu.SemaphoreType.DMA] * 6,
    )
    return f(rowf, col3, normf, rh, zeros)


def _rh_body(x_r, h_r, ax_r, ah_r, wr0a_r, wr0b_r, wr1a_r, wr1b_r, br_r,
             rh_r):
    dot = functools.partial(jnp.dot, preferred_element_type=jnp.float32)
    gr = (dot(x_r[...], wr0a_r[...]) + dot(h_r[...], wr0b_r[...])
          + dot(ax_r[...], wr1a_r[...]) + dot(ah_r[...], wr1b_r[...])
          + br_r[...])
    rh_r[...] = jax.nn.sigmoid(gr) * h_r[...]


def _rh_tc(x, h, a_cat, wr0a, wr0b, wr1a, wr1b, br2):
    nb = 10
    rs = pl.BlockSpec((N // nb, C), lambda i: (i, 0))
    rs2 = pl.BlockSpec((N // nb, C), lambda i: (i + nb, 0))
    ws = pl.BlockSpec((C, C), lambda i: (0, 0))
    bs = pl.BlockSpec((1, C), lambda i: (0, 0))
    return pl.pallas_call(
        _rh_body,
        grid=(nb,),
        in_specs=[rs, rs, rs, rs2, ws, ws, ws, ws, bs],
        out_specs=rs,
        out_shape=jax.ShapeDtypeStruct((N, C), jnp.float32),
    )(x, h, a_cat, a_cat, wr0a, wr0b, wr1a, wr1b, br2)


def _out_body(x_r, h_r, rh_r, ax_r, ah_r, ar0_r, ar1_r,
              wz0a_r, wz0b_r, wz1a_r, wz1b_r, bz_r,
              wh0a_r, wh0b_r, wh1a_r, wh1b_r, bh_r, o_r):
    dot = functools.partial(jnp.dot, preferred_element_type=jnp.float32)
    xx, hh, ax = x_r[...], h_r[...], ax_r[...]
    gz = (dot(xx, wz0a_r[...]) + dot(hh, wz0b_r[...])
          + dot(ax, wz1a_r[...]) + dot(ah_r[...], wz1b_r[...]) + bz_r[...])
    z = jax.nn.sigmoid(gz)
    cand = jnp.tanh(dot(xx, wh0a_r[...]) + dot(rh_r[...], wh0b_r[...])
                    + dot(ax, wh1a_r[...])
                    + dot(ar0_r[...] + ar1_r[...], wh1b_r[...]) + bh_r[...])
    o_r[...] = (1.0 - z) * hh + z * cand


def _out_tc(x, h, rh, a_cat, ar, wz0a, wz0b, wz1a, wz1b, bz2,
            wh0a, wh0b, wh1a, wh1b, bh2):
    nb = 10
    rs = pl.BlockSpec((N // nb, C), lambda i: (i, 0))
    rs2 = pl.BlockSpec((N // nb, C), lambda i: (i + nb, 0))
    ws = pl.BlockSpec((C, C), lambda i: (0, 0))
    bs = pl.BlockSpec((1, C), lambda i: (0, 0))
    return pl.pallas_call(
        _out_body,
        grid=(nb,),
        in_specs=[rs, rs, rs, rs, rs2, rs, rs2,
                  ws, ws, ws, ws, bs, ws, ws, ws, ws, bs],
        out_specs=rs,
        out_shape=jax.ShapeDtypeStruct((N, C), jnp.float32),
    )(x, h, rh, a_cat, a_cat, ar, ar,
      wz0a, wz0b, wz1a, wz1b, bz2, wh0a, wh0b, wh1a, wh1b, bh2)


def kernel(x, edge_index, edge_weight, hidden_state,
           Wz0, Wz1, bz, Wr0, Wr1, br, Wh0, Wh1, bh):
    row = edge_index[0]
    col = edge_index[1]
    zeros = jnp.zeros((128, C), jnp.float32)
    xh = jnp.concatenate([x, hidden_state], axis=0)

    rowf = row.reshape(NBLK, EPB)
    row3 = row.reshape(NBLK, BCH, CH)
    col3 = col.reshape(NBLK, BCH, CH)
    ewf = edge_weight.reshape(NBLK, EPB)
    a_cat, normf = _agg_xh(rowf, row3, col3, ewf, xh, zeros)

    rh = _rh_tc(x, hidden_state, a_cat,
                Wr0[:C], Wr0[C:], Wr1[:C], Wr1[C:], br.reshape(1, C))

    ar = _agg_rh(rowf, col3, normf, rh, zeros)

    return _out_tc(x, hidden_state, rh, a_cat, ar,
                   Wz0[:C], Wz0[C:], Wz1[:C], Wz1[C:], bz.reshape(1, C),
                   Wh0[:C], Wh0[C:], Wh1[:C], Wh1[C:], bh.reshape(1, C))


# speedup vs baseline: 1.3335x; 1.3335x over previous
import functools

import jax
import jax.numpy as jnp
from jax import lax
from jax.experimental import pallas as pl
from jax.experimental.pallas import tpu as pltpu
from jax.experimental.pallas import tpu_sc as plsc

N = 10000
E = 320000
C = 128
NPAD = 10240
NC = 2
NS = 16
NW = NC * NS
CH = 80
NROWS = E // CH
L = 16
NBUF = 5

_mesh = lambda: plsc.VectorSubcoreMesh(core_axis_name="c", subcore_axis_name="s")
_params = lambda: pltpu.CompilerParams(needs_layout_passes=False)


def _rsqrt_newton(d):
    i = plsc.bitcast(d, jnp.int32)
    y = plsc.bitcast(jnp.int32(0x5F3759DF) - lax.shift_right_logical(i, 1),
                     jnp.float32)
    for _ in range(3):
        y = y * (1.5 - 0.5 * d * y * y)
    return jnp.where(d > 0, y, 0.0)


BCH = 25
EPB = BCH * CH
NBLK = E // EPB


def _agg_pipeline(tab_h, rowb_v, col2_v, normb_v, bufs, gsems, ssems, acc_s):
    def fire_gather(q, b):
        pltpu.async_copy(tab_h.at[rowb_v.at[pl.ds(q * CH, CH)]], bufs[b],
                         gsems[b])

    def wait_gather(q, b):
        pltpu.make_async_copy(tab_h.at[rowb_v.at[pl.ds(q * CH, CH)]],
                              bufs[b], gsems[b]).wait()

    def wait_scatter(q, b):
        pltpu.make_async_copy(bufs[b], acc_s.at[col2_v.at[q]],
                              ssems[b]).wait()

    fire_gather(0, 0)
    fire_gather(1, 1)

    def slot(q, b, g):
        wait_gather(q, b)

        def scale(g2, _):
            nv = normb_v[pl.ds(q * CH + g2 * L, L)]
            for e in range(L):
                sv = jnp.full((L,), nv[e], jnp.float32)
                r = g2 * L + e
                for u in range(C // L):
                    bufs[b][r, pl.ds(u * L, L)] = (
                        bufs[b][r, pl.ds(u * L, L)] * sv)
            return 0
        lax.fori_loop(0, CH // L, scale, 0)

        pltpu.async_copy(bufs[b], acc_s.at[col2_v.at[q]], ssems[b], add=True)

        bg = (b + 2) % 3

        def refill():
            wait_scatter(q, bg)
            fire_gather(q + 2, bg)

        if b == 0:
            @pl.when(g > 0)
            def _():
                refill()

            @pl.when(g == 0)
            def _():
                fire_gather(q + 2, bg)
        elif b == 1:
            refill()
        else:
            @pl.when(g < BCH // 3 - 1)
            def _():
                refill()

    def outer_body(g, _):
        for b in range(3):
            slot(g * 3 + b, b, g)
        return 0
    lax.fori_loop(0, BCH // 3, outer_body, 0)

    slot_q = BCH - 1
    wait_gather(slot_q, 0)

    def scale_t(g2, _):
        nv = normb_v[pl.ds(slot_q * CH + g2 * L, L)]
        for e in range(L):
            sv = jnp.full((L,), nv[e], jnp.float32)
            r = g2 * L + e
            for u in range(C // L):
                bufs[0][r, pl.ds(u * L, L)] = bufs[0][r, pl.ds(u * L, L)] * sv
        return 0
    lax.fori_loop(0, CH // L, scale_t, 0)
    pltpu.async_copy(bufs[0], acc_s.at[col2_v.at[slot_q]], ssems[0], add=True)

    for b in range(3):
        wait_scatter(0, b)


def _agg_xh_kernel(rowf_h, row3_h, col3_h, ewf_h, xh_h, zeros_h,
                   a_out_h, norm_out_h,
                   dis_v, rowb_v, col2_v, normb_v,
                   b0, b1, b2,
                   acc_s, deg_s,
                   g0, g1, g2, s0, s1, s2, dsem):
    c = lax.axis_index("c")
    s = lax.axis_index("s")
    bufs = (b0, b1, b2)
    gsems = (g0, g1, g2)
    ssems = (s0, s1, s2)

    def zb(i, _):
        normb_v[pl.ds(i * L, L)] = jnp.zeros((L,), jnp.float32)
        return 0
    lax.fori_loop(0, 640 // L, zb, 0)

    @pl.when(s < 15)
    def _():
        pltpu.sync_copy(normb_v.at[pl.ds(0, 640)],
                        deg_s.at[pl.ds(s * 640, 640)])
        for i in range(5):
            pltpu.sync_copy(zeros_h, acc_s.at[pl.ds(s * 640 + i * 128, 128)])

    @pl.when(s == 15)
    def _():
        pltpu.sync_copy(normb_v.at[pl.ds(0, 400)],
                        deg_s.at[pl.ds(9600, 400)])
        for i in range(3):
            pltpu.sync_copy(zeros_h, acc_s.at[pl.ds(9600 + i * 128, 128)])
        pltpu.sync_copy(zeros_h.at[pl.ds(0, 16)],
                        acc_s.at[pl.ds(9984, 16)])
    plsc.subcore_barrier()

    def deg_blk(kb, _):
        k2 = s * (2 * NBLK // NW) + kb
        pltpu.async_copy(row3_h.at[k2], col2_v, g0)
        pltpu.async_copy(ewf_h.at[k2], normb_v, g1)
        pltpu.make_async_copy(row3_h.at[k2], col2_v, g0).wait()
        pltpu.make_async_copy(ewf_h.at[k2], normb_v, g1).wait()

        def dfire(q, _):
            pltpu.async_copy(normb_v.at[pl.ds(q * CH, CH)],
                             deg_s.at[col2_v.at[q]], dsem, add=True)
            return 0
        lax.fori_loop(0, BCH, dfire, 0)

        def ddrain(q, _):
            pltpu.make_async_copy(normb_v.at[pl.ds(q * CH, CH)],
                                  deg_s.at[col2_v.at[q]], dsem).wait()
            return 0
        lax.fori_loop(0, BCH, ddrain, 0)
        return 0
    lax.fori_loop(0, 2 * NBLK // NW, deg_blk, 0)
    plsc.subcore_barrier()

    def newton(i, _):
        d = normb_v[pl.ds(i * L, L)]
        normb_v[pl.ds(i * L, L)] = _rsqrt_newton(d)
        return 0

    @pl.when(s < 15)
    def _():
        pltpu.sync_copy(deg_s.at[pl.ds(s * 640, 640)],
                        normb_v.at[pl.ds(0, 640)])
        lax.fori_loop(0, 640 // L, newton, 0)
        pltpu.sync_copy(normb_v.at[pl.ds(0, 640)],
                        deg_s.at[pl.ds(s * 640, 640)])

    @pl.when(s == 15)
    def _():
        pltpu.sync_copy(deg_s.at[pl.ds(9600, 400)],
                        normb_v.at[pl.ds(0, 400)])
        lax.fori_loop(0, 400 // L, newton, 0)
        pltpu.sync_copy(normb_v.at[pl.ds(0, 400)],
                        deg_s.at[pl.ds(9600, 400)])
    plsc.subcore_barrier()
    pltpu.sync_copy(deg_s, dis_v)

    off = jnp.full((L,), c * N, jnp.int32)

    def main_blk(kb, _):
        k2 = s * (2 * NBLK // NW) + kb
        pltpu.async_copy(rowf_h.at[k2], rowb_v, g0)
        pltpu.async_copy(col3_h.at[k2], col2_v, g1)
        pltpu.async_copy(ewf_h.at[k2], normb_v, g2)
        pltpu.make_async_copy(rowf_h.at[k2], rowb_v, g0).wait()
        pltpu.make_async_copy(col3_h.at[k2], col2_v, g1).wait()
        pltpu.make_async_copy(ewf_h.at[k2], normb_v, g2).wait()

        def normk(i, _):
            r16 = rowb_v[pl.ds(i * L, L)]
            c16 = col2_v[i // (CH // L), pl.ds((i % (CH // L)) * L, L)]
            w16 = normb_v[pl.ds(i * L, L)]
            a = plsc.load_gather(dis_v, [r16])
            b = plsc.load_gather(dis_v, [c16])
            normb_v[pl.ds(i * L, L)] = -(a * w16) * b
            rowb_v[pl.ds(i * L, L)] = r16 + off
            return 0
        lax.fori_loop(0, EPB // L, normk, 0)

        @pl.when(c == 0)
        def _():
            pltpu.sync_copy(normb_v, norm_out_h.at[k2])

        _agg_pipeline(xh_h, rowb_v, col2_v, normb_v, bufs, gsems, ssems,
                      acc_s)
        return 0
    lax.fori_loop(0, 2 * NBLK // NW, main_blk, 0)

    plsc.subcore_barrier()

    @pl.when(s < 15)
    def _():
        pltpu.sync_copy(acc_s.at[pl.ds(s * 640, 640)],
                        a_out_h.at[pl.ds(c * N + s * 640, 640)])

    @pl.when(s == 15)
    def _():
        pltpu.sync_copy(acc_s.at[pl.ds(9600, 400)],
                        a_out_h.at[pl.ds(c * N + 9600, 400)])


def _agg_xh(rowf, row3, col3, ewf, xh, zeros):
    f = pl.kernel(
        _agg_xh_kernel,
        out_type=(jax.ShapeDtypeStruct((2 * N, C), jnp.float32),
                  jax.ShapeDtypeStruct((NBLK, EPB), jnp.float32)),
        mesh=_mesh(),
        compiler_params=_params(),
        scratch_types=[
            pltpu.VMEM((N,), jnp.float32),
            pltpu.VMEM((EPB,), jnp.int32),
            pltpu.VMEM((BCH, CH), jnp.int32),
            pltpu.VMEM((EPB,), jnp.float32),
        ] + [pltpu.VMEM((CH, C), jnp.float32)] * 3
        + [pltpu.VMEM_SHARED((N, C), jnp.float32),
           pltpu.VMEM_SHARED((N,), jnp.float32)]
        + [pltpu.SemaphoreType.DMA] * 7,
    )
    return f(rowf, row3, col3, ewf, xh, zeros)


def _agg_rh_kernel(rowf_h, col3_h, normf_h, rh_h, zeros_h, a_out_h,
                   rowb_v, col2_v, normb_v,
                   b0, b1, b2,
                   acc_s,
                   g0, g1, g2, s0, s1, s2):
    c = lax.axis_index("c")
    s = lax.axis_index("s")
    w = c * NS + s
    bufs = (b0, b1, b2)
    gsems = (g0, g1, g2)
    ssems = (s0, s1, s2)

    @pl.when(s < 15)
    def _():
        for i in range(5):
            pltpu.sync_copy(zeros_h, acc_s.at[pl.ds(s * 640 + i * 128, 128)])

    @pl.when(s == 15)
    def _():
        for i in range(3):
            pltpu.sync_copy(zeros_h, acc_s.at[pl.ds(9600 + i * 128, 128)])
        pltpu.sync_copy(zeros_h.at[pl.ds(0, 16)],
                        acc_s.at[pl.ds(9984, 16)])
    plsc.subcore_barrier()

    def main_blk(kb, _):
        k2 = w * (NBLK // NW) + kb
        pltpu.async_copy(rowf_h.at[k2], rowb_v, g0)
        pltpu.async_copy(col3_h.at[k2], col2_v, g1)
        pltpu.async_copy(normf_h.at[k2], normb_v, g2)
        pltpu.make_async_copy(rowf_h.at[k2], rowb_v, g0).wait()
        pltpu.make_async_copy(col3_h.at[k2], col2_v, g1).wait()
        pltpu.make_async_copy(normf_h.at[k2], normb_v, g2).wait()
        _agg_pipeline(rh_h, rowb_v, col2_v, normb_v, bufs, gsems, ssems,
                      acc_s)
        return 0
    lax.fori_loop(0, NBLK // NW, main_blk, 0)

    plsc.subcore_barrier()

    @pl.when(s < 15)
    def _():
        pltpu.sync_copy(acc_s.at[pl.ds(s * 640, 640)],
                        a_out_h.at[pl.ds(c * N + s * 640, 640)])

    @pl.when(s == 15)
    def _():
        pltpu.sync_copy(acc_s.at[pl.ds(9600, 400)],
                        a_out_h.at[pl.ds(c * N + 9600, 400)])


def _agg_rh(rowf, col3, normf, rh, zeros):
    f = pl.kernel(
        _agg_rh_kernel,
        out_type=jax.ShapeDtypeStruct((2 * N, C), jnp.float32),
        mesh=_mesh(),
        compiler_params=_params(),
        scratch_types=[
            pltpu.VMEM((EPB,), jnp.int32),
            pltpu.VMEM((BCH, CH), jnp.int32),
            pltpu.VMEM((EPB,), jnp.float32),
        ] + [pltpu.VMEM((CH, C), jnp.float32)] * 3
        + [pltpu.VMEM_SHARED((N, C), jnp.float32)]
        + [pltpu.SemaphoreType.DMA] * 6,
    )
    return f(rowf, col3, normf, rh, zeros)


def _rh_body(x_r, h_r, ax_r, ah_r, wr0a_r, wr0b_r, wr1a_r, wr1b_r, br_r,
             rh_r):
    dot = functools.partial(jnp.dot, preferred_element_type=jnp.float32)
    gr = (dot(x_r[...], wr0a_r[...]) + dot(h_r[...], wr0b_r[...])
          + dot(ax_r[...], wr1a_r[...]) + dot(ah_r[...], wr1b_r[...])
          + br_r[...])
    rh_r[...] = jax.nn.sigmoid(gr) * h_r[...]


def _rh_tc(x, h, a_cat, wr0a, wr0b, wr1a, wr1b, br2):
    nb = 10
    rs = pl.BlockSpec((N // nb, C), lambda i: (i, 0))
    rs2 = pl.BlockSpec((N // nb, C), lambda i: (i + nb, 0))
    ws = pl.BlockSpec((C, C), lambda i: (0, 0))
    bs = pl.BlockSpec((1, C), lambda i: (0, 0))
    return pl.pallas_call(
        _rh_body,
        grid=(nb,),
        in_specs=[rs, rs, rs, rs2, ws, ws, ws, ws, bs],
        out_specs=rs,
        out_shape=jax.ShapeDtypeStruct((N, C), jnp.float32),
    )(x, h, a_cat, a_cat, wr0a, wr0b, wr1a, wr1b, br2)


def _out_body(x_r, h_r, rh_r, ax_r, ah_r, ar0_r, ar1_r,
              wz0a_r, wz0b_r, wz1a_r, wz1b_r, bz_r,
              wh0a_r, wh0b_r, wh1a_r, wh1b_r, bh_r, o_r):
    dot = functools.partial(jnp.dot, preferred_element_type=jnp.float32)
    xx, hh, ax = x_r[...], h_r[...], ax_r[...]
    gz = (dot(xx, wz0a_r[...]) + dot(hh, wz0b_r[...])
          + dot(ax, wz1a_r[...]) + dot(ah_r[...], wz1b_r[...]) + bz_r[...])
    z = jax.nn.sigmoid(gz)
    cand = jnp.tanh(dot(xx, wh0a_r[...]) + dot(rh_r[...], wh0b_r[...])
                    + dot(ax, wh1a_r[...])
                    + dot(ar0_r[...] + ar1_r[...], wh1b_r[...]) + bh_r[...])
    o_r[...] = (1.0 - z) * hh + z * cand


def _out_tc(x, h, rh, a_cat, ar, wz0a, wz0b, wz1a, wz1b, bz2,
            wh0a, wh0b, wh1a, wh1b, bh2):
    nb = 10
    rs = pl.BlockSpec((N // nb, C), lambda i: (i, 0))
    rs2 = pl.BlockSpec((N // nb, C), lambda i: (i + nb, 0))
    ws = pl.BlockSpec((C, C), lambda i: (0, 0))
    bs = pl.BlockSpec((1, C), lambda i: (0, 0))
    return pl.pallas_call(
        _out_body,
        grid=(nb,),
        in_specs=[rs, rs, rs, rs, rs2, rs, rs2,
                  ws, ws, ws, ws, bs, ws, ws, ws, ws, bs],
        out_specs=rs,
        out_shape=jax.ShapeDtypeStruct((N, C), jnp.float32),
    )(x, h, rh, a_cat, a_cat, ar, ar,
      wz0a, wz0b, wz1a, wz1b, bz2, wh0a, wh0b, wh1a, wh1b, bh2)


def kernel(x, edge_index, edge_weight, hidden_state,
           Wz0, Wz1, bz, Wr0, Wr1, br, Wh0, Wh1, bh):
    row = edge_index[0]
    col = edge_index[1]
    zeros = jnp.zeros((128, C), jnp.float32)
    xh = jnp.concatenate([x, hidden_state], axis=0)

    rowf = row.reshape(NBLK, EPB)
    row3 = row.reshape(NBLK, BCH, CH)
    col3 = col.reshape(NBLK, BCH, CH)
    ewf = edge_weight.reshape(NBLK, EPB)
    a_cat, normf = _agg_xh(rowf, row3, col3, ewf, xh, zeros)

    rh = _rh_tc(x, hidden_state, a_cat,
                Wr0[:C], Wr0[C:], Wr1[:C], Wr1[C:], br.reshape(1, C))

    ar = _agg_rh(rowf, col3, normf, rh, zeros)

    return _out_tc(x, hidden_state, rh, a_cat, ar,
                   Wz0[:C], Wz0[C:], Wz1[:C], Wz1[C:], bz.reshape(1, C),
                   Wh0[:C], Wh0[C:], Wh1[:C], Wh1[C:], bh.reshape(1, C))
